# baseline (speedup 1.0000x reference)
import jax
import jax.numpy as jnp
from jax import lax
from jax.experimental import pallas as pl
from jax.experimental.pallas import tpu as pltpu
from jax.experimental.pallas import tpu_sc as plsc

N_USER = 25000
N_ITEM = 25000
N_DAYDIFF = 5
N_BIGCAT = 9
N_TAGS = 912
N_TESTIDS = 1537
TAB_REP = 4
N_TAGS = 912
N = N_USER + N_ITEM
D = 64
E = 800000
NC, NS, L = 2, 16, 16
NW = NC * NS
BLK = 128

A_BLOCKS = (N_USER + BLK - 1) // BLK
A_PER_W = (A_BLOCKS + NW - 1) // NW
A_LAST_START = A_BLOCKS - A_PER_W
A_ROW_LAST = N_USER - BLK

E_BLOCKS = E // BLK
B_PER_W = (E_BLOCKS + NW - 1) // NW
B_LAST_START = E_BLOCKS - B_PER_W

S_USER = 1.0 / 6.0
S_ITEM = 1.0 / 12.0


def _worker_id():
    return lax.axis_index("s") * NC + lax.axis_index("c")


_HI_MASK = -65536


def _pack_rows(rows_v, prow_v):
    def body(r, carry):
        for h in (0, 16):
            va = rows_v[r, pl.ds(h, L)]
            vb = rows_v[r, pl.ds(32 + h, L)]
            ua = plsc.bitcast(va, jnp.int32) + 0x8000
            ub = plsc.bitcast(vb, jnp.int32) + 0x8000
            lo = lax.shift_right_logical(ua, 16)
            prow_v[r, pl.ds(h, L)] = lo | (ub & _HI_MASK)
        return carry

    lax.fori_loop(0, BLK, body, 0)


def _small_table_add(rows_v, idx_v, tab_v):
    skew = lax.iota(jnp.int32, L)

    def body(g, carry):
        rowv = skew + g * L
        idxv = idx_v[pl.ds(g * L, L)]
        for c in range(D):
            col = (skew + c) & (D - 1)
            val = plsc.load_gather(tab_v, [idxv, col])
            plsc.addupdate_scatter(rows_v, [rowv, col], val)
        return carry

    lax.fori_loop(0, BLK // L, body, 0)


def _build_x_body(user_emb, item_emb, day_diff, ktag, tid, bcat,
                  dd_emb, tag_emb, tid_emb, bc_emb, x_out,
                  urows0, urows1, irows0, irows1,
                  uprow0, uprow1, iprow0, iprow1,
                  uidx0, uidx1, kidx0, kidx1, tidx0, tidx1, bidx0, bidx1,
                  ddv, bcv,
                  s1a, s1b, s2a, s2b, s3a, s3b):
    wid = _worker_id()
    jstart = jnp.minimum(wid * A_PER_W, A_LAST_START)
    slots = [
        dict(urows=urows0, irows=irows0, uprow=uprow0, iprow=iprow0,
             uidx=uidx0, kidx=kidx0, tidx=tidx0, bidx=bidx0,
             sem1=s1a, sem2=s2a, sem3=s3a),
        dict(urows=urows1, irows=irows1, uprow=uprow1, iprow=iprow1,
             uidx=uidx1, kidx=kidx1, tidx=tidx1, bidx=bidx1,
             sem1=s1b, sem2=s2b, sem3=s3b),
    ]

    pltpu.sync_copy(dd_emb, ddv)
    pltpu.sync_copy(bc_emb, bcv)

    def row_start(b):
        return pl.multiple_of(jnp.minimum((jstart + b) * BLK, A_ROW_LAST), 8)

    def fire_stage1(b, S):
        rs = row_start(b)
        ac = pltpu.async_copy
        return [
            ac(day_diff.at[pl.ds(rs, BLK)], S["uidx"], S["sem1"]),
            ac(user_emb.at[pl.ds(rs, BLK)], S["urows"], S["sem1"]),
            ac(ktag.at[pl.ds(rs, BLK)], S["kidx"], S["sem1"]),
            ac(tid.at[pl.ds(rs, BLK)], S["tidx"], S["sem1"]),
            ac(bcat.at[pl.ds(rs, BLK)], S["bidx"], S["sem1"]),
            ac(item_emb.at[pl.ds(rs, BLK)], S["irows"], S["sem1"]),
        ]

    d_stage1 = {0: fire_stage1(0, slots[0])}
    d_wb = {}
    ac = pltpu.async_copy

    koff = (wid % TAB_REP) * N_TAGS
    toff = (wid % TAB_REP) * N_TESTIDS

    def add_offset(idx_ref, off):
        offv = jnp.full((L,), off, jnp.int32)

        def body(g, carry):
            idx_ref[pl.ds(g * L, L)] = idx_ref[pl.ds(g * L, L)] + offv
            return carry

        lax.fori_loop(0, BLK // L, body, 0)

    def do_adds(b):
        S = slots[b % 2]
        for dsc in d_stage1[b % 2]:
            dsc.wait()
        add_offset(S["kidx"], koff)
        add_offset(S["tidx"], toff)
        d = [
            ac(tag_emb.at[S["kidx"]], S["irows"], S["sem2"], add=True),
            ac(tid_emb.at[S["tidx"]], S["irows"], S["sem2"], add=True),
        ]
        _small_table_add(S["urows"], S["uidx"], ddv)
        return d

    def finish_adds(b, d):
        S = slots[b % 2]
        for dsc in d:
            dsc.wait()
        _small_table_add(S["irows"], S["bidx"], bcv)

    def pack_and_wb(b):
        s = b % 2
        S = slots[s]
        if b >= 2:
            for dsc in d_wb[s]:
                dsc.wait()
        _pack_rows(S["urows"], S["uprow"])
        _pack_rows(S["irows"], S["iprow"])
        rs = row_start(b)
        d_wb[s] = [
            ac(S["uprow"], x_out.at[pl.ds(rs, BLK)], S["sem3"]),
            ac(S["iprow"], x_out.at[pl.ds(N_USER + rs, BLK)], S["sem3"]),
        ]

    pend = do_adds(0)
    for b in range(A_PER_W):
        if b >= 1:
            pack_and_wb(b - 1)
        if b + 1 < A_PER_W:
            d_stage1[(b + 1) % 2] = fire_stage1(b + 1, slots[(b + 1) % 2])
        finish_adds(b, pend)
        if b + 1 < A_PER_W:
            pend = do_adds(b + 1)
    pack_and_wb(A_PER_W - 1)
    for s in d_wb:
        for dsc in d_wb[s]:
            dsc.wait()


B_HALF = B_PER_W // 2


def _edge_dot_body(x_hbm, eidx_hbm, out_hbm,
                   sidx, didx,
                   srow0, drow0, srow1, drow1, srow2, drow2, srow3, drow3,
                   outv,
                   ss0, sd0, ss1, sd1, ss2, sd2, ss3, sd3):
    wid = _worker_id()
    bstart = jnp.minimum(wid * B_PER_W, B_LAST_START)
    estart = pl.multiple_of(bstart * BLK, 8 * BLK)
    pltpu.sync_copy(eidx_hbm.at[0, pl.ds(estart, B_PER_W * BLK)], sidx)
    pltpu.sync_copy(eidx_hbm.at[1, pl.ds(estart, B_PER_W * BLK)], didx)
    srows = (srow0, srow1, srow2, srow3)
    drows = (drow0, drow1, drow2, drow3)
    sems_s = (ss0, ss1, ss2, ss3)
    sems_d = (sd0, sd1, sd2, sd3)

    def fire(t, s):
        tt = jnp.minimum(t, B_PER_W - 1) * BLK
        pltpu.async_copy(x_hbm.at[sidx.at[pl.ds(tt, BLK)]], srows[s], sems_s[s])
        pltpu.async_copy(x_hbm.at[didx.at[pl.ds(tt, BLK)]], drows[s], sems_d[s])

    def drain(s):
        pltpu.make_async_copy(
            x_hbm.at[sidx.at[pl.ds(0, BLK)]], srows[s], sems_s[s]).wait()
        pltpu.make_async_copy(
            x_hbm.at[didx.at[pl.ds(0, BLK)]], drows[s], sems_d[s]).wait()

    def compute(t, s):
        srow, drow = srows[s], drows[s]

        def group(g, carry):
            rowid = lax.iota(jnp.int32, L) + (g * L)
            si = sidx[pl.ds(t * BLK + g * L, L)]
            di = didx[pl.ds(t * BLK + g * L, L)]
            f = (jnp.where(si < N_USER, jnp.float32(S_USER), jnp.float32(S_ITEM))
                 * jnp.where(di < N_USER, jnp.float32(S_USER), jnp.float32(S_ITEM)))
            accs = [jnp.zeros((L,), jnp.float32) for _ in range(4)]
            skew = lax.iota(jnp.int32, L)
            for k in range(D // 2):
                col = (skew + k) & (D // 2 - 1)
                wa = plsc.load_gather(srow, [rowid, col])
                wb = plsc.load_gather(drow, [rowid, col])
                prod = plsc.bitcast(wa, jnp.bfloat16) * plsc.bitcast(wb, jnp.bfloat16)
                u = plsc.bitcast(prod, jnp.int32)
                plo = plsc.bitcast(lax.shift_left(u, 16), jnp.float32)
                phi = plsc.bitcast(u & _HI_MASK, jnp.float32)
                accs[2 * (k % 2)] = accs[2 * (k % 2)] + plo
                accs[2 * (k % 2) + 1] = accs[2 * (k % 2) + 1] + phi
            acc = (accs[0] + accs[1]) + (accs[2] + accs[3])
            outv[pl.ds(t * BLK + g * L, L)] = acc * f
            return carry

        lax.fori_loop(0, BLK // L, group, 0)

    fire(0, 0)
    fire(1, 1)
    fire(2, 2)

    def step(i, carry):
        for j in range(4):
            t = 4 * i + j
            fire(t + 3, (j + 3) % 4)
            drain(j)
            compute(t, j)
        return carry

    lax.fori_loop(0, B_PER_W // 4, step, 0)
    for s in range(3):
        drain(s)
    pltpu.sync_copy(outv, out_hbm.at[pl.ds(estart, B_PER_W * BLK)])


def kernel(edge_index, knowledge_tag, test_id, big_category, day_diff,
           edge_weight, user_emb, item_emb, tag_emb, testid_emb,
           bigcat_emb, daydiff_emb):
    del edge_weight
    tag_rep = jnp.tile(tag_emb, (TAB_REP, 1))
    tid_rep = jnp.tile(testid_emb, (TAB_REP, 1))
    mesh = plsc.VectorSubcoreMesh(core_axis_name="c", subcore_axis_name="s")
    params = pltpu.CompilerParams(use_tc_tiling_on_sc=False,
                                  needs_layout_passes=False)

    x = pl.kernel(
        _build_x_body,
        out_type=jax.ShapeDtypeStruct((N, D // 2), jnp.int32),
        mesh=mesh,
        scratch_types=(
            [pltpu.VMEM((BLK, D), jnp.float32)] * 4
            + [pltpu.VMEM((BLK, D // 2), jnp.int32)] * 4
            + [pltpu.VMEM((BLK,), jnp.int32)] * 8
            + [pltpu.VMEM((N_DAYDIFF, D), jnp.float32),
               pltpu.VMEM((N_BIGCAT, D), jnp.float32)]
            + [pltpu.SemaphoreType.DMA] * 6
        ),
        compiler_params=params,
    )(user_emb, item_emb, day_diff, knowledge_tag, test_id, big_category,
      daydiff_emb, tag_rep, tid_rep, bigcat_emb)

    out = pl.kernel(
        _edge_dot_body,
        out_type=jax.ShapeDtypeStruct((E,), jnp.float32),
        mesh=mesh,
        scratch_types=[
            pltpu.VMEM((B_PER_W * BLK,), jnp.int32),
            pltpu.VMEM((B_PER_W * BLK,), jnp.int32),
        ] + [pltpu.VMEM((BLK, D // 2), jnp.int32)] * 8 + [
            pltpu.VMEM((B_PER_W * BLK,), jnp.float32),
        ] + [pltpu.SemaphoreType.DMA] * 8,
        compiler_params=params,
    )(x, edge_index)

    return out


# speedup vs baseline: 1.0041x; 1.0041x over previous
import jax
import jax.numpy as jnp
from jax import lax
from jax.experimental import pallas as pl
from jax.experimental.pallas import tpu as pltpu
from jax.experimental.pallas import tpu_sc as plsc

N_USER = 25000
N_ITEM = 25000
N_DAYDIFF = 5
N_BIGCAT = 9
N_TAGS = 912
N = N_USER + N_ITEM
D = 64
E = 800000
NC, NS, L = 2, 16, 16
NW = NC * NS
BLK = 128

A_BLOCKS = (N_USER + BLK - 1) // BLK
A_PER_W = (A_BLOCKS + NW - 1) // NW
A_LAST_START = A_BLOCKS - A_PER_W
A_ROW_LAST = N_USER - BLK

E_BLOCKS = E // BLK
B_PER_W = (E_BLOCKS + NW - 1) // NW
B_LAST_START = E_BLOCKS - B_PER_W

S_USER = 1.0 / 6.0
S_ITEM = 1.0 / 12.0


def _worker_id():
    return lax.axis_index("s") * NC + lax.axis_index("c")


_HI_MASK = -65536


def _pack_rows(rows_v, prow_v):
    def body(r, carry):
        for h in (0, 16):
            va = rows_v[r, pl.ds(h, L)]
            vb = rows_v[r, pl.ds(32 + h, L)]
            ua = plsc.bitcast(va, jnp.int32) + 0x8000
            ub = plsc.bitcast(vb, jnp.int32) + 0x8000
            lo = lax.shift_right_logical(ua, 16)
            prow_v[r, pl.ds(h, L)] = lo | (ub & _HI_MASK)
        return carry

    lax.fori_loop(0, BLK, body, 0)


def _small_table_add(rows_v, idx_v, tab_v):
    skew = lax.iota(jnp.int32, L)

    def body(g, carry):
        rowv = skew + g * L
        idxv = idx_v[pl.ds(g * L, L)]
        for c in range(D):
            col = (skew + c) & (D - 1)
            val = plsc.load_gather(tab_v, [idxv, col])
            plsc.addupdate_scatter(rows_v, [rowv, col], val)
        return carry

    lax.fori_loop(0, BLK // L, body, 0)


def _build_x_body(user_emb, item_emb, day_diff, ktag, tid, bcat,
                  dd_emb, tag_emb, tid_emb, bc_emb, x_out,
                  urows0, urows1, irows0, irows1,
                  uprow0, uprow1, iprow0, iprow1,
                  uidx0, uidx1, kidx0, kidx1, tidx0, tidx1, bidx0, bidx1,
                  ddv, bcv,
                  s1a, s1b, s2a, s2b, s3a, s3b):
    wid = _worker_id()
    jstart = jnp.minimum(wid * A_PER_W, A_LAST_START)
    slots = [
        dict(urows=urows0, irows=irows0, uprow=uprow0, iprow=iprow0,
             uidx=uidx0, kidx=kidx0, tidx=tidx0, bidx=bidx0,
             sem1=s1a, sem2=s2a, sem3=s3a),
        dict(urows=urows1, irows=irows1, uprow=uprow1, iprow=iprow1,
             uidx=uidx1, kidx=kidx1, tidx=tidx1, bidx=bidx1,
             sem1=s1b, sem2=s2b, sem3=s3b),
    ]

    pltpu.sync_copy(dd_emb, ddv)
    pltpu.sync_copy(bc_emb, bcv)

    def row_start(b):
        return pl.multiple_of(jnp.minimum((jstart + b) * BLK, A_ROW_LAST), 8)

    def fire_stage1(b, S):
        rs = row_start(b)
        ac = pltpu.async_copy
        return [
            ac(day_diff.at[pl.ds(rs, BLK)], S["uidx"], S["sem1"]),
            ac(user_emb.at[pl.ds(rs, BLK)], S["urows"], S["sem1"]),
            ac(ktag.at[pl.ds(rs, BLK)], S["kidx"], S["sem1"]),
            ac(tid.at[pl.ds(rs, BLK)], S["tidx"], S["sem1"]),
            ac(bcat.at[pl.ds(rs, BLK)], S["bidx"], S["sem1"]),
            ac(item_emb.at[pl.ds(rs, BLK)], S["irows"], S["sem1"]),
        ]

    d_stage1 = {0: fire_stage1(0, slots[0])}
    d_wb = {}
    ac = pltpu.async_copy

    def do_adds(b):
        S = slots[b % 2]
        for dsc in d_stage1[b % 2]:
            dsc.wait()
        d = [
            ac(tag_emb.at[S["kidx"]], S["irows"], S["sem2"], add=True),
            ac(tid_emb.at[S["tidx"]], S["irows"], S["sem2"], add=True),
        ]
        _small_table_add(S["urows"], S["uidx"], ddv)
        return d

    def finish_adds(b, d):
        S = slots[b % 2]
        for dsc in d:
            dsc.wait()
        _small_table_add(S["irows"], S["bidx"], bcv)

    def pack_and_wb(b):
        s = b % 2
        S = slots[s]
        if b >= 2:
            for dsc in d_wb[s]:
                dsc.wait()
        _pack_rows(S["urows"], S["uprow"])
        _pack_rows(S["irows"], S["iprow"])
        rs = row_start(b)
        d_wb[s] = [
            ac(S["uprow"], x_out.at[pl.ds(rs, BLK)], S["sem3"]),
            ac(S["iprow"], x_out.at[pl.ds(N_USER + rs, BLK)], S["sem3"]),
        ]

    pend = do_adds(0)
    for b in range(A_PER_W):
        if b >= 1:
            pack_and_wb(b - 1)
        if b + 1 < A_PER_W:
            d_stage1[(b + 1) % 2] = fire_stage1(b + 1, slots[(b + 1) % 2])
        finish_adds(b, pend)
        if b + 1 < A_PER_W:
            pend = do_adds(b + 1)
    pack_and_wb(A_PER_W - 1)
    for s in d_wb:
        for dsc in d_wb[s]:
            dsc.wait()


B_HALF = B_PER_W // 2


def _edge_dot_body(x_hbm, eidx_hbm, out_hbm,
                   sidx, didx,
                   srow0, drow0, srow1, drow1, srow2, drow2, srow3, drow3,
                   outv,
                   ss0, sd0, ss1, sd1, ss2, sd2, ss3, sd3):
    wid = _worker_id()
    bstart = jnp.minimum(wid * B_PER_W, B_LAST_START)
    estart = pl.multiple_of(bstart * BLK, 8 * BLK)
    pltpu.sync_copy(eidx_hbm.at[0, pl.ds(estart, B_PER_W * BLK)], sidx)
    pltpu.sync_copy(eidx_hbm.at[1, pl.ds(estart, B_PER_W * BLK)], didx)
    srows = (srow0, srow1, srow2, srow3)
    drows = (drow0, drow1, drow2, drow3)
    sems_s = (ss0, ss1, ss2, ss3)
    sems_d = (sd0, sd1, sd2, sd3)

    def fire(t, s):
        tt = jnp.minimum(t, B_PER_W - 1) * BLK
        pltpu.async_copy(x_hbm.at[sidx.at[pl.ds(tt, BLK)]], srows[s], sems_s[s])
        pltpu.async_copy(x_hbm.at[didx.at[pl.ds(tt, BLK)]], drows[s], sems_d[s])

    def drain(s):
        pltpu.make_async_copy(
            x_hbm.at[sidx.at[pl.ds(0, BLK)]], srows[s], sems_s[s]).wait()
        pltpu.make_async_copy(
            x_hbm.at[didx.at[pl.ds(0, BLK)]], drows[s], sems_d[s]).wait()

    def compute(t, s):
        srow, drow = srows[s], drows[s]

        def group(g, carry):
            rowid = lax.iota(jnp.int32, L) + (g * L)
            si = sidx[pl.ds(t * BLK + g * L, L)]
            di = didx[pl.ds(t * BLK + g * L, L)]
            f = (jnp.where(si < N_USER, jnp.float32(S_USER), jnp.float32(S_ITEM))
                 * jnp.where(di < N_USER, jnp.float32(S_USER), jnp.float32(S_ITEM)))
            accs = [jnp.zeros((L,), jnp.float32) for _ in range(4)]
            skew = lax.iota(jnp.int32, L)
            for k in range(D // 2):
                col = (skew + k) & (D // 2 - 1)
                wa = plsc.load_gather(srow, [rowid, col])
                wb = plsc.load_gather(drow, [rowid, col])
                prod = plsc.bitcast(wa, jnp.bfloat16) * plsc.bitcast(wb, jnp.bfloat16)
                u = plsc.bitcast(prod, jnp.int32)
                plo = plsc.bitcast(lax.shift_left(u, 16), jnp.float32)
                phi = plsc.bitcast(u & _HI_MASK, jnp.float32)
                accs[2 * (k % 2)] = accs[2 * (k % 2)] + plo
                accs[2 * (k % 2) + 1] = accs[2 * (k % 2) + 1] + phi
            acc = (accs[0] + accs[1]) + (accs[2] + accs[3])
            outv[pl.ds(t * BLK + g * L, L)] = acc * f
            return carry

        lax.fori_loop(0, BLK // L, group, 0)

    fire(0, 0)
    fire(1, 1)
    fire(2, 2)

    def step(i, carry):
        for j in range(4):
            t = 4 * i + j
            fire(t + 3, (j + 3) % 4)
            drain(j)
            compute(t, j)
        return carry

    lax.fori_loop(0, B_PER_W // 4, step, 0)
    for s in range(3):
        drain(s)
    pltpu.sync_copy(outv, out_hbm.at[pl.ds(estart, B_PER_W * BLK)])


def kernel(edge_index, knowledge_tag, test_id, big_category, day_diff,
           edge_weight, user_emb, item_emb, tag_emb, testid_emb,
           bigcat_emb, daydiff_emb):
    del edge_weight
    mesh = plsc.VectorSubcoreMesh(core_axis_name="c", subcore_axis_name="s")
    params = pltpu.CompilerParams(use_tc_tiling_on_sc=False,
                                  needs_layout_passes=False)

    x = pl.kernel(
        _build_x_body,
        out_type=jax.ShapeDtypeStruct((N, D // 2), jnp.int32),
        mesh=mesh,
        scratch_types=(
            [pltpu.VMEM((BLK, D), jnp.float32)] * 4
            + [pltpu.VMEM((BLK, D // 2), jnp.int32)] * 4
            + [pltpu.VMEM((BLK,), jnp.int32)] * 8
            + [pltpu.VMEM((N_DAYDIFF, D), jnp.float32),
               pltpu.VMEM((N_BIGCAT, D), jnp.float32)]
            + [pltpu.SemaphoreType.DMA] * 6
        ),
        compiler_params=params,
    )(user_emb, item_emb, day_diff, knowledge_tag, test_id, big_category,
      daydiff_emb, tag_emb, testid_emb, bigcat_emb)

    out = pl.kernel(
        _edge_dot_body,
        out_type=jax.ShapeDtypeStruct((E,), jnp.float32),
        mesh=mesh,
        scratch_types=[
            pltpu.VMEM((B_PER_W * BLK,), jnp.int32),
            pltpu.VMEM((B_PER_W * BLK,), jnp.int32),
        ] + [pltpu.VMEM((BLK, D // 2), jnp.int32)] * 8 + [
            pltpu.VMEM((B_PER_W * BLK,), jnp.float32),
        ] + [pltpu.SemaphoreType.DMA] * 8,
        compiler_params=params,
    )(x, edge_index)

    return out
